# baseline (speedup 1.0000x reference)
import functools
import math

import jax
import jax.numpy as jnp
from jax import lax
from jax.experimental import pallas as pl
from jax.experimental.pallas import tpu as pltpu
from jax.experimental.pallas import tpu_sc as plsc

NC = 2
NS = 16
NW = NC * NS
LANES = 16

CHUNK = 128
NBUF = 4


@functools.cache
def _build(n_s: int, d: int):
    mesh = plsc.VectorSubcoreMesh(core_axis_name="c", subcore_axis_name="s")
    dh = d // 8

    scratch = (
        [pltpu.VMEM((n_s, CHUNK), jnp.int32)]
        + [pltpu.VMEM((CHUNK, d), jnp.float32) for _ in range(NBUF)]
        + [pltpu.VMEM((dh, 8, CHUNK), jnp.float32) for _ in range(NBUF)]
        + [pltpu.SemaphoreType.DMA for _ in range(2 * NBUF)]
    )

    @functools.partial(
        pl.kernel,
        out_type=jax.ShapeDtypeStruct((n_s, dh, NW, 8, CHUNK), jnp.float32),
        mesh=mesh,
        scratch_types=scratch,
        compiler_params=pltpu.CompilerParams(
            use_tc_tiling_on_sc=False, needs_layout_passes=False
        ),
    )
    def emb(table_hbm, idx_hbm, out_hbm, *scr):
        idx_v = scr[0]
        gbuf = scr[1:1 + NBUF]
        obuf = scr[1 + NBUF:1 + 2 * NBUF]
        gsem = scr[1 + 2 * NBUF:1 + 3 * NBUF]
        osem = scr[1 + 3 * NBUF:1 + 4 * NBUF]

        w = lax.axis_index("c") * NS + lax.axis_index("s")

        pltpu.sync_copy(idx_hbm.at[:, pl.ds(w * CHUNK, CHUNK)], idx_v)

        lane = lax.iota(jnp.int32, LANES)
        rows = [lane + (qlb * LANES) for qlb in range(CHUNK // LANES)]

        def start_gather(s, b):
            pltpu.make_async_copy(
                table_hbm.at[idx_v.at[s]], gbuf[b], gsem[b]
            ).start()

        def transpose_scale(b):
            def col(dd, _):
                hi = dd // 8
                lo = dd % 8
                cidx = jnp.full((LANES,), dd, jnp.int32)
                for qlb in range(CHUNK // LANES):
                    v = plsc.load_gather(gbuf[b], [rows[qlb], cidx])
                    obuf[b][hi, lo, pl.ds(qlb * LANES, LANES)] = v * 8.0
                return 0

            lax.fori_loop(0, d, col, 0, unroll=2)

        def step(s, b, wait_out, start_next):
            pltpu.make_async_copy(
                table_hbm.at[idx_v.at[s]], gbuf[b], gsem[b]
            ).wait()
            if wait_out:
                pltpu.make_async_copy(
                    obuf[b], out_hbm.at[s, :, w], osem[b]
                ).wait()
            transpose_scale(b)
            pltpu.make_async_copy(obuf[b], out_hbm.at[s, :, w], osem[b]).start()
            if start_next:
                start_gather(s + NBUF, b)

        n_outer = n_s // NBUF

        for b in range(NBUF):
            start_gather(b, b)

        for b in range(NBUF):
            step(b, b, wait_out=False, start_next=True)

        def outer(o, _):
            for b in range(NBUF):
                step(o * NBUF + b, b, wait_out=True, start_next=True)
            return 0

        lax.fori_loop(1, n_outer - 1, outer, 0)

        for b in range(NBUF):
            step((n_outer - 1) * NBUF + b, b, wait_out=True, start_next=False)

        for b in range(NBUF):
            pltpu.make_async_copy(
                obuf[b], out_hbm.at[(n_outer - 1) * NBUF + b, :, w], osem[b]
            ).wait()

    return emb


def kernel(x, table):
    d = table.shape[1]
    q, n_s = x.shape
    assert q == NW * CHUNK and d % 8 == 0
    idx_t = jnp.transpose(x).astype(jnp.int32)
    out5 = _build(n_s, d)(table, idx_t)
    return jnp.transpose(out5, (2, 4, 0, 1, 3)).reshape(q, n_s, d)


# speedup vs baseline: 1.3865x; 1.3865x over previous
import functools
import math

import jax
import jax.numpy as jnp
from jax import lax
from jax.experimental import pallas as pl
from jax.experimental.pallas import tpu as pltpu
from jax.experimental.pallas import tpu_sc as plsc

NC = 2
NS = 16
NW = NC * NS
LANES = 16

CHUNK = 128
NBUF = 4


@functools.cache
def _build(n_s: int, d: int):
    mesh = plsc.VectorSubcoreMesh(core_axis_name="c", subcore_axis_name="s")
    dh = d // 8

    scratch = (
        [pltpu.VMEM((n_s, CHUNK), jnp.int32)]
        + [pltpu.VMEM((CHUNK, d), jnp.float32) for _ in range(NBUF)]
        + [pltpu.VMEM((dh, 8, CHUNK), jnp.float32) for _ in range(NBUF)]
        + [pltpu.SemaphoreType.DMA for _ in range(2 * NBUF)]
    )

    @functools.partial(
        pl.kernel,
        out_type=jax.ShapeDtypeStruct((n_s, dh, NW, 8, CHUNK), jnp.float32),
        mesh=mesh,
        scratch_types=scratch,
        compiler_params=pltpu.CompilerParams(
            use_tc_tiling_on_sc=False, needs_layout_passes=False
        ),
    )
    def emb(table_hbm, idx_hbm, out_hbm, *scr):
        idx_v = scr[0]
        gbuf = scr[1:1 + NBUF]
        obuf = scr[1 + NBUF:1 + 2 * NBUF]
        gsem = scr[1 + 2 * NBUF:1 + 3 * NBUF]
        osem = scr[1 + 3 * NBUF:1 + 4 * NBUF]

        w = lax.axis_index("c") * NS + lax.axis_index("s")

        pltpu.sync_copy(idx_hbm.at[:, pl.ds(w * CHUNK, CHUNK)], idx_v)

        lane = lax.iota(jnp.int32, LANES)
        rows = [lane + (qlb * LANES) for qlb in range(CHUNK // LANES)]

        def start_gather(s, b):
            pltpu.make_async_copy(
                table_hbm.at[idx_v.at[s]], gbuf[b], gsem[b]
            ).start()

        def transpose_scale(b):
            @plsc.parallel_loop(0, d // 8, 1)
            def _ts(hi):
                for lo in range(8):
                    cidx = jnp.full((LANES,), hi * 8 + lo, jnp.int32)
                    for qlb in range(CHUNK // LANES):
                        v = plsc.load_gather(gbuf[b], [rows[qlb], cidx])
                        obuf[b][hi, lo, pl.ds(qlb * LANES, LANES)] = v * 8.0

        n_outer = n_s // NBUF

        for b in range(NBUF):
            start_gather(b, b)

        def outer(o, _):
            for b in range(NBUF):
                s = o * NBUF + b
                pltpu.make_async_copy(
                    table_hbm.at[idx_v.at[s]], gbuf[b], gsem[b]
                ).wait()

                @pl.when(o > 0)
                def _():
                    pltpu.make_async_copy(
                        obuf[b], out_hbm.at[s, :, w], osem[b]
                    ).wait()

                transpose_scale(b)
                pltpu.make_async_copy(
                    obuf[b], out_hbm.at[s, :, w], osem[b]
                ).start()

                @pl.when(o < n_outer - 1)
                def _():
                    start_gather(s + NBUF, b)

            return 0

        lax.fori_loop(0, n_outer, outer, 0)

        for b in range(NBUF):
            pltpu.make_async_copy(
                obuf[b], out_hbm.at[(n_outer - 1) * NBUF + b, :, w], osem[b]
            ).wait()

    return emb


def kernel(x, table):
    d = table.shape[1]
    q, n_s = x.shape
    assert q == NW * CHUNK and d % 8 == 0
    idx_t = jnp.transpose(x).astype(jnp.int32)
    out5 = _build(n_s, d)(table, idx_t)
    return jnp.transpose(out5, (2, 4, 0, 1, 3)).reshape(q, n_s, d)


# speedup vs baseline: 1.4909x; 1.0753x over previous
import functools
import math

import jax
import jax.numpy as jnp
from jax import lax
from jax.experimental import pallas as pl
from jax.experimental.pallas import tpu as pltpu
from jax.experimental.pallas import tpu_sc as plsc

NC = 2
NS = 16
NW = NC * NS
LANES = 16

CHUNK = 128
NBUF = 4


@functools.cache
def _build(n_s: int, d: int):
    mesh = plsc.VectorSubcoreMesh(core_axis_name="c", subcore_axis_name="s")
    dh = d // 8
    ngrp = d // LANES

    scratch = (
        [pltpu.VMEM((n_s, CHUNK), jnp.int32)]
        + [pltpu.VMEM((CHUNK, d), jnp.float32) for _ in range(NBUF)]
        + [pltpu.VMEM((dh, 8 * CHUNK), jnp.float32) for _ in range(NBUF)]
        + [pltpu.SemaphoreType.DMA for _ in range(2 * NBUF)]
    )

    @functools.partial(
        pl.kernel,
        out_type=jax.ShapeDtypeStruct((n_s, dh, NW, 8 * CHUNK), jnp.float32),
        mesh=mesh,
        scratch_types=scratch,
        compiler_params=pltpu.CompilerParams(
            use_tc_tiling_on_sc=False, needs_layout_passes=False
        ),
    )
    def emb(table_hbm, idx_hbm, out_hbm, *scr):
        idx_v = scr[0]
        gbuf = scr[1:1 + NBUF]
        obuf = scr[1 + NBUF:1 + 2 * NBUF]
        gsem = scr[1 + 2 * NBUF:1 + 3 * NBUF]
        osem = scr[1 + 3 * NBUF:1 + 4 * NBUF]

        w = lax.axis_index("c") * NS + lax.axis_index("s")

        pltpu.sync_copy(idx_hbm.at[:, pl.ds(w * CHUNK, CHUNK)], idx_v)

        lane = lax.iota(jnp.int32, LANES)
        hi_idx = [(g * LANES + lane) // 8 for g in range(ngrp)]
        lo_off = [((g * LANES + lane) % 8) * CHUNK for g in range(ngrp)]

        def start_gather(s, b):
            pltpu.make_async_copy(
                table_hbm.at[idx_v.at[s]], gbuf[b], gsem[b]
            ).start()

        def transpose_scale(b):
            @plsc.parallel_loop(0, CHUNK, 1, unroll=4)
            def _ts(r):
                for g in range(ngrp):
                    v = gbuf[b][r, pl.ds(g * LANES, LANES)]
                    plsc.store_scatter(
                        obuf[b], [hi_idx[g], lo_off[g] + r], v * 8.0
                    )

        n_outer = n_s // NBUF

        for b in range(NBUF):
            start_gather(b, b)

        def outer(o, _):
            for b in range(NBUF):
                s = o * NBUF + b
                pltpu.make_async_copy(
                    table_hbm.at[idx_v.at[s]], gbuf[b], gsem[b]
                ).wait()

                @pl.when(o > 0)
                def _():
                    pltpu.make_async_copy(
                        obuf[b], out_hbm.at[s, :, w], osem[b]
                    ).wait()

                transpose_scale(b)
                pltpu.make_async_copy(
                    obuf[b], out_hbm.at[s, :, w], osem[b]
                ).start()

                @pl.when(o < n_outer - 1)
                def _():
                    start_gather(s + NBUF, b)

            return 0

        lax.fori_loop(0, n_outer, outer, 0)

        for b in range(NBUF):
            pltpu.make_async_copy(
                obuf[b], out_hbm.at[(n_outer - 1) * NBUF + b, :, w], osem[b]
            ).wait()

    return emb


def kernel(x, table):
    d = table.shape[1]
    q, n_s = x.shape
    assert q == NW * CHUNK and d % 8 == 0
    idx_t = jnp.transpose(x).astype(jnp.int32)
    t2 = jnp.reshape(table, (table.shape[0] // 2, 2 * d))
    t2 = jax.lax.optimization_barrier(t2)
    t_lin = jnp.reshape(t2, table.shape)
    out4 = _build(n_s, d)(t_lin, idx_t)
    out5 = out4.reshape(n_s, d // 8, NW, 8, CHUNK)
    return jnp.transpose(out5, (2, 4, 0, 1, 3)).reshape(q, n_s, d)


# speedup vs baseline: 2.5991x; 1.7433x over previous
import functools
import math

import jax
import jax.numpy as jnp
from jax import lax
from jax.experimental import pallas as pl
from jax.experimental.pallas import tpu as pltpu
from jax.experimental.pallas import tpu_sc as plsc

NC = 2
NS = 16
NW = NC * NS
LANES = 16

CHUNK = 128
NBUF = 4


@functools.cache
def _build(n_s: int, d: int):
    mesh = plsc.VectorSubcoreMesh(core_axis_name="c", subcore_axis_name="s")
    dh = d // 8
    ngrp = d // LANES

    PADC = CHUNK + 1

    scratch = (
        [pltpu.VMEM((n_s, CHUNK), jnp.int32)]
        + [pltpu.VMEM((CHUNK, d), jnp.float32) for _ in range(NBUF)]
        + [pltpu.VMEM((dh, 8, PADC), jnp.float32) for _ in range(NBUF)]
        + [pltpu.SemaphoreType.DMA for _ in range(2 * NBUF)]
    )

    @functools.partial(
        pl.kernel,
        out_type=jax.ShapeDtypeStruct((n_s, dh, NW, 8, CHUNK), jnp.float32),
        mesh=mesh,
        scratch_types=scratch,
        compiler_params=pltpu.CompilerParams(
            use_tc_tiling_on_sc=False, needs_layout_passes=False
        ),
    )
    def emb(table_hbm, idx_hbm, out_hbm, *scr):
        idx_v = scr[0]
        gbuf = scr[1:1 + NBUF]
        obuf = scr[1 + NBUF:1 + 2 * NBUF]
        gsem = scr[1 + 2 * NBUF:1 + 3 * NBUF]
        osem = scr[1 + 3 * NBUF:1 + 4 * NBUF]

        w = lax.axis_index("c") * NS + lax.axis_index("s")

        pltpu.sync_copy(idx_hbm.at[:, pl.ds(w * CHUNK, CHUNK)], idx_v)

        lane = lax.iota(jnp.int32, LANES)
        hi_idx = [(g * LANES + lane) // 8 for g in range(ngrp)]
        mid_idx = [(g * LANES + lane) % 8 for g in range(ngrp)]

        def start_gather(s, b):
            pltpu.make_async_copy(
                table_hbm.at[idx_v.at[s]], gbuf[b], gsem[b]
            ).start()

        def transpose_scale(b):
            @plsc.parallel_loop(0, CHUNK, 1, unroll=4)
            def _ts(r):
                rvec = jnp.full((LANES,), r, jnp.int32)
                for g in range(ngrp):
                    v = gbuf[b][r, pl.ds(g * LANES, LANES)]
                    plsc.store_scatter(
                        obuf[b], [hi_idx[g], mid_idx[g], rvec], v * 8.0
                    )

        n_outer = n_s // NBUF

        for b in range(NBUF):
            start_gather(b, b)

        def outer(o, _):
            for b in range(NBUF):
                s = o * NBUF + b
                pltpu.make_async_copy(
                    table_hbm.at[idx_v.at[s]], gbuf[b], gsem[b]
                ).wait()

                @pl.when(o > 0)
                def _():
                    pltpu.make_async_copy(
                        obuf[b].at[:, :, pl.ds(0, CHUNK)],
                        out_hbm.at[s, :, w], osem[b]
                    ).wait()

                transpose_scale(b)
                pltpu.make_async_copy(
                    obuf[b].at[:, :, pl.ds(0, CHUNK)],
                    out_hbm.at[s, :, w], osem[b]
                ).start()

                @pl.when(o < n_outer - 1)
                def _():
                    start_gather(s + NBUF, b)

            return 0

        lax.fori_loop(0, n_outer, outer, 0)

        for b in range(NBUF):
            pltpu.make_async_copy(
                obuf[b].at[:, :, pl.ds(0, CHUNK)],
                out_hbm.at[(n_outer - 1) * NBUF + b, :, w], osem[b]
            ).wait()

    return emb


def kernel(x, table):
    d = table.shape[1]
    q, n_s = x.shape
    assert q == NW * CHUNK and d % 8 == 0
    idx_t = jnp.transpose(x).astype(jnp.int32)
    t2 = jnp.reshape(table, (table.shape[0] // 2, 2 * d))
    t2 = jax.lax.optimization_barrier(t2)
    t_lin = jnp.reshape(t2, table.shape)
    out5 = _build(n_s, d)(t_lin, idx_t)
    return jnp.transpose(out5, (2, 4, 0, 1, 3)).reshape(q, n_s, d)
